# baseline (device time: 82004 ns/iter reference)
import jax
import jax.numpy as jnp
from jax import lax
from jax.experimental import pallas as pl
from jax.experimental.pallas import tpu as pltpu

N_DEV = 4


def kernel(x, w_mat, scale_x, scale_w):
    m_total, k_per = x.shape
    _, n = w_mat.shape
    m_per = m_total // N_DEV

    w_mat = w_mat.astype(jnp.float8_e4m3fn)

    def body(x_ref, w_ref, sx_ref, sw_ref, out_ref,
             xsl, x8send, send_p8, recv_xblk, recv_w, recv_p8,
             lsems, send_sems, recv_sems):
        my = lax.axis_index("i")
        left = lax.rem(my + N_DEV - 1, N_DEV)
        right = lax.rem(my + 1, N_DEV)
        diag = lax.rem(my + 2, N_DEV)

        cp_x0 = pltpu.make_async_copy(
            x_ref.at[pl.ds(left * m_per, m_per), :], xsl.at[0], lsems.at[0])
        cp_x0.start()
        cp_x1 = pltpu.make_async_copy(
            x_ref.at[pl.ds(right * m_per, m_per), :], xsl.at[1], lsems.at[1])
        cp_x1.start()

        barrier = pltpu.get_barrier_semaphore()
        for j in range(1, N_DEV):
            peer = lax.rem(my + j, N_DEV)
            pl.semaphore_signal(
                barrier, inc=1,
                device_id=(peer,), device_id_type=pl.DeviceIdType.MESH,
            )
        pl.semaphore_wait(barrier, N_DEV - 1)

        sends = []
        wm = pltpu.make_async_remote_copy(
            src_ref=w_ref,
            dst_ref=recv_w,
            send_sem=send_sems.at[0],
            recv_sem=recv_sems.at[0],
            device_id=(left,),
            device_id_type=pl.DeviceIdType.MESH,
        )
        wm.start()
        sends.append(wm)
        cp_x0.wait()
        x8send[...] = xsl[0].astype(jnp.float8_e4m3fn)
        xb = pltpu.make_async_remote_copy(
            src_ref=x8send,
            dst_ref=recv_xblk,
            send_sem=send_sems.at[1],
            recv_sem=recv_sems.at[1],
            device_id=(left,),
            device_id_type=pl.DeviceIdType.MESH,
        )
        xb.start()
        sends.append(xb)

        cp_x2 = pltpu.make_async_copy(
            x_ref.at[pl.ds(diag * m_per, m_per), :], xsl.at[0], lsems.at[2])
        cp_x2.start()

        w = w_ref[...].astype(jnp.bfloat16)

        for idx, (tgt, cp, slot) in enumerate(
                ((right, cp_x1, 1), (diag, cp_x2, 0))):
            cp.wait()
            chunk = lax.dot_general(
                xsl[slot].astype(jnp.bfloat16), w, (((1,), (0,)), ((), ())),
                preferred_element_type=jnp.float32,
            )
            send_p8[idx, :, :] = chunk.astype(jnp.float8_e4m3fn)
            dg = pltpu.make_async_remote_copy(
                src_ref=send_p8.at[idx],
                dst_ref=recv_p8.at[idx],
                send_sem=send_sems.at[2 + idx],
                recv_sem=recv_sems.at[2 + idx],
                device_id=(tgt,),
                device_id_type=pl.DeviceIdType.MESH,
            )
            dg.start()
            sends.append(dg)
            if idx == 0:
                cp_own = pltpu.make_async_copy(
                    x_ref.at[pl.ds(my * m_per, m_per), :], xsl.at[1],
                    lsems.at[3])
                cp_own.start()

        cp_own.wait()
        acc = lax.dot_general(
            xsl[1].astype(jnp.bfloat16), w, (((1,), (0,)), ((), ())),
            preferred_element_type=jnp.float32,
        )

        for sem, dst in ((0, recv_w), (1, recv_xblk)):
            recv = pltpu.make_async_remote_copy(
                src_ref=dst,
                dst_ref=dst,
                send_sem=send_sems.at[0],
                recv_sem=recv_sems.at[sem],
                device_id=(my,),
                device_id_type=pl.DeviceIdType.MESH,
            )
            recv.wait_recv()
        acc = acc + lax.dot_general(
            recv_xblk[...].astype(jnp.bfloat16),
            recv_w[...].astype(jnp.bfloat16),
            (((1,), (0,)), ((), ())),
            preferred_element_type=jnp.float32,
        )

        for idx in range(2):
            recv = pltpu.make_async_remote_copy(
                src_ref=recv_p8.at[idx],
                dst_ref=recv_p8.at[idx],
                send_sem=send_sems.at[0],
                recv_sem=recv_sems.at[2 + idx],
                device_id=(my,),
                device_id_type=pl.DeviceIdType.MESH,
            )
            recv.wait_recv()
            acc = acc + recv_p8[idx, :, :].astype(jnp.float32)

        scale = sx_ref[0] * sw_ref[0]
        out_ref[...] = jnp.maximum(acc * scale, 0.0)

        for rdma in sends:
            rdma.wait_send()

    return pl.pallas_call(
        body,
        out_shape=jax.ShapeDtypeStruct((m_per, n), jnp.float32),
        in_specs=[
            pl.BlockSpec(memory_space=pltpu.HBM),
            pl.BlockSpec(memory_space=pltpu.VMEM),
            pl.BlockSpec(memory_space=pltpu.SMEM),
            pl.BlockSpec(memory_space=pltpu.SMEM),
        ],
        out_specs=pl.BlockSpec(memory_space=pltpu.VMEM),
        scratch_shapes=[
            pltpu.VMEM((2, m_per, k_per), jnp.float32),
            pltpu.VMEM((m_per, k_per), jnp.float8_e4m3fn),
            pltpu.VMEM((2, m_per, n), jnp.float8_e4m3fn),
            pltpu.VMEM((m_per, k_per), jnp.float8_e4m3fn),
            pltpu.VMEM((k_per, n), jnp.float8_e4m3fn),
            pltpu.VMEM((2, m_per, n), jnp.float8_e4m3fn),
            pltpu.SemaphoreType.DMA((4,)),
            pltpu.SemaphoreType.DMA((4,)),
            pltpu.SemaphoreType.DMA((4,)),
        ],
        compiler_params=pltpu.CompilerParams(
            collective_id=0,
            vmem_limit_bytes=62 * 1024 * 1024,
        ),
    )(x, w_mat, scale_x, scale_w)


# device time: 79317 ns/iter; 1.0339x vs baseline; 1.0339x over previous
import jax
import jax.numpy as jnp
from jax import lax
from jax.experimental import pallas as pl
from jax.experimental.pallas import tpu as pltpu

N_DEV = 4


def kernel(x, w_mat, scale_x, scale_w):
    m_total, k_per = x.shape
    _, n = w_mat.shape
    m_per = m_total // N_DEV

    def body(x_ref, w_ref, sx_ref, sw_ref, out_ref,
             xsl, wsl, w8send, x8send, send_p8, recv_xblk, recv_w, recv_p8,
             lsems, send_sems, recv_sems):
        my = lax.axis_index("i")
        left = lax.rem(my + N_DEV - 1, N_DEV)
        right = lax.rem(my + 1, N_DEV)
        diag = lax.rem(my + 2, N_DEV)

        cp_w = pltpu.make_async_copy(w_ref, wsl, lsems.at[4])
        cp_w.start()
        cp_x0 = pltpu.make_async_copy(
            x_ref.at[pl.ds(right * m_per, m_per), :], xsl.at[0], lsems.at[0])
        cp_x0.start()
        cp_x1 = pltpu.make_async_copy(
            x_ref.at[pl.ds(left * m_per, m_per), :], xsl.at[1], lsems.at[1])
        cp_x1.start()

        barrier = pltpu.get_barrier_semaphore()
        for j in range(1, N_DEV):
            peer = lax.rem(my + j, N_DEV)
            pl.semaphore_signal(
                barrier, inc=1,
                device_id=(peer,), device_id_type=pl.DeviceIdType.MESH,
            )
        pl.semaphore_wait(barrier, N_DEV - 1)

        sends = []
        for idx, (nbr, slot, cp) in enumerate(
                ((right, 0, cp_x0), (left, 1, cp_x1))):
            cp.wait()
            x8send[idx, :, :] = xsl[idx].astype(jnp.float8_e4m3fn)
            xb = pltpu.make_async_remote_copy(
                src_ref=x8send.at[idx],
                dst_ref=recv_xblk.at[slot],
                send_sem=send_sems.at[2 * idx],
                recv_sem=recv_sems.at[slot],
                device_id=(nbr,),
                device_id_type=pl.DeviceIdType.MESH,
            )
            xb.start()
            sends.append(xb)
        cp_w.wait()
        w8send[...] = wsl[...].astype(jnp.float8_e4m3fn)
        for idx, (nbr, slot) in enumerate(((right, 0), (left, 1))):
            wm = pltpu.make_async_remote_copy(
                src_ref=w8send,
                dst_ref=recv_w.at[slot],
                send_sem=send_sems.at[2 * idx + 1],
                recv_sem=recv_sems.at[2 + slot],
                device_id=(nbr,),
                device_id_type=pl.DeviceIdType.MESH,
            )
            wm.start()
            sends.append(wm)

        cp_x2 = pltpu.make_async_copy(
            x_ref.at[pl.ds(diag * m_per, m_per), :], xsl.at[0], lsems.at[2])
        cp_x2.start()
        cp_x3 = pltpu.make_async_copy(
            x_ref.at[pl.ds(my * m_per, m_per), :], xsl.at[1], lsems.at[3])
        cp_x3.start()

        w = wsl[...].astype(jnp.bfloat16)

        cp_x2.wait()
        chunk = lax.dot_general(
            xsl[0].astype(jnp.bfloat16), w, (((1,), (0,)), ((), ())),
            preferred_element_type=jnp.float32,
        )
        send_p8[...] = chunk.astype(jnp.float8_e4m3fn)
        dg = pltpu.make_async_remote_copy(
            src_ref=send_p8,
            dst_ref=recv_p8,
            send_sem=send_sems.at[4],
            recv_sem=recv_sems.at[4],
            device_id=(diag,),
            device_id_type=pl.DeviceIdType.MESH,
        )
        dg.start()
        sends.append(dg)

        cp_x3.wait()
        acc = lax.dot_general(
            xsl[1].astype(jnp.bfloat16), w, (((1,), (0,)), ((), ())),
            preferred_element_type=jnp.float32,
        )

        for slot in range(2):
            for sem, dst in ((slot, recv_xblk.at[slot]),
                             (2 + slot, recv_w.at[slot])):
                recv = pltpu.make_async_remote_copy(
                    src_ref=dst,
                    dst_ref=dst,
                    send_sem=send_sems.at[0],
                    recv_sem=recv_sems.at[sem],
                    device_id=(my,),
                    device_id_type=pl.DeviceIdType.MESH,
                )
                recv.wait_recv()
            acc = acc + lax.dot_general(
                recv_xblk[slot].astype(jnp.bfloat16),
                recv_w[slot].astype(jnp.bfloat16),
                (((1,), (0,)), ((), ())),
                preferred_element_type=jnp.float32,
            )

        recv = pltpu.make_async_remote_copy(
            src_ref=recv_p8,
            dst_ref=recv_p8,
            send_sem=send_sems.at[0],
            recv_sem=recv_sems.at[4],
            device_id=(my,),
            device_id_type=pl.DeviceIdType.MESH,
        )
        recv.wait_recv()
        acc = acc + recv_p8[...].astype(jnp.float32)

        scale = sx_ref[0] * sw_ref[0]
        out_ref[...] = jnp.maximum(acc * scale, 0.0)

        for rdma in sends:
            rdma.wait_send()

    return pl.pallas_call(
        body,
        out_shape=jax.ShapeDtypeStruct((m_per, n), jnp.float32),
        in_specs=[
            pl.BlockSpec(memory_space=pltpu.HBM),
            pl.BlockSpec(memory_space=pltpu.HBM),
            pl.BlockSpec(memory_space=pltpu.SMEM),
            pl.BlockSpec(memory_space=pltpu.SMEM),
        ],
        out_specs=pl.BlockSpec(memory_space=pltpu.VMEM),
        scratch_shapes=[
            pltpu.VMEM((2, m_per, k_per), jnp.float32),
            pltpu.VMEM((k_per, n), jnp.float32),
            pltpu.VMEM((k_per, n), jnp.float8_e4m3fn),
            pltpu.VMEM((2, m_per, k_per), jnp.float8_e4m3fn),
            pltpu.VMEM((m_per, n), jnp.float8_e4m3fn),
            pltpu.VMEM((2, m_per, k_per), jnp.float8_e4m3fn),
            pltpu.VMEM((2, k_per, n), jnp.float8_e4m3fn),
            pltpu.VMEM((m_per, n), jnp.float8_e4m3fn),
            pltpu.SemaphoreType.DMA((5,)),
            pltpu.SemaphoreType.DMA((5,)),
            pltpu.SemaphoreType.DMA((5,)),
        ],
        compiler_params=pltpu.CompilerParams(
            collective_id=0,
            vmem_limit_bytes=62 * 1024 * 1024,
        ),
    )(x, w_mat, scale_x, scale_w)
